# baseline (speedup 1.0000x reference)
import functools

import jax
import jax.numpy as jnp
from jax import lax
from jax.experimental import pallas as pl
from jax.experimental.pallas import tpu as pltpu
from jax.experimental.pallas import tpu_sc as plsc

_NUM_CORES = 2
_NUM_SUBCORES = 16
_NUM_WORKERS = _NUM_CORES * _NUM_SUBCORES

_LANES = 16


def _worker_id():
    return lax.axis_index("s") * _NUM_CORES + lax.axis_index("c")


def _format_table(tT, *, blk, unroll):
    D, V = tT.shape
    total_blocks = V // blk
    assert total_blocks * blk == V and D == 2 * _LANES and blk % 8 == 0
    base_blocks, extra = divmod(total_blocks, _NUM_WORKERS)

    mesh = plsc.VectorSubcoreMesh(core_axis_name="c", subcore_axis_name="s")

    @functools.partial(
        pl.kernel,
        out_type=jax.ShapeDtypeStruct((V, D), jnp.float32),
        mesh=mesh,
        compiler_params=pltpu.CompilerParams(use_tc_tiling_on_sc=False, needs_layout_passes=False),
        scratch_types=[
            pltpu.VMEM((2, D, blk), jnp.float32),
            pltpu.VMEM((2, blk, D), jnp.float32),
            pltpu.SemaphoreType.DMA((2,)),
            pltpu.SemaphoreType.DMA((2,)),
        ],
    )
    def fmt(tT_hbm, tL_hbm, in_v, out_v, isem, osem):
        wid = _worker_id()
        n_blocks = base_blocks + jnp.where(wid < extra, 1, 0)

        def col0(j):
            return pl.multiple_of((wid + j * _NUM_WORKERS) * blk, 8)

        def copy_in(j, b):
            return pltpu.make_async_copy(
                tT_hbm.at[:, pl.ds(col0(j), blk)], in_v.at[b],
                isem.at[b])

        def copy_out(j, b):
            return pltpu.make_async_copy(
                out_v.at[b], tL_hbm.at[pl.ds(col0(j), blk)],
                osem.at[b])

        copy_in(0, 0).start()
        iota = lax.iota(jnp.int32, _LANES)

        def body(j, carry):
            b = lax.rem(j, 2)
            nb = lax.rem(j + 1, 2)

            @pl.when(j + 1 < n_blocks)
            def _prefetch():
                copy_in(j + 1, nb).start()

            copy_in(j, b).wait()

            @pl.when(j >= 2)
            def _drain():
                copy_out(j - 2, b).wait()

            def rows(rr, c2):
                for u in range(unroll):
                    r = rr * unroll + u
                    r_idx = jnp.zeros((_LANES,), jnp.int32) + r
                    for half in range(2):
                        c_idx = iota + (half * _LANES)
                        vals = plsc.load_gather(in_v.at[b], [c_idx, r_idx])
                        out_v[b, r, pl.ds(half * _LANES, _LANES)] = vals
                return c2

            lax.fori_loop(0, blk // unroll, rows, 0)
            copy_out(j, b).start()
            return carry

        lax.fori_loop(0, n_blocks, body, 0)
        copy_out(n_blocks - 2, lax.rem(n_blocks - 2, 2)).wait()
        copy_out(n_blocks - 1, lax.rem(n_blocks - 1, 2)).wait()

    return fmt(tT)


def _gather_t(xT, tL):
    H, B = xT.shape
    V, D = tL.shape
    BW = B // _NUM_WORKERS
    assert D == 2 * _LANES

    mesh = plsc.VectorSubcoreMesh(core_axis_name="c", subcore_axis_name="s")

    @functools.partial(
        pl.kernel,
        out_type=jax.ShapeDtypeStruct((H, D, B), jnp.float32),
        mesh=mesh,
        compiler_params=pltpu.CompilerParams(use_tc_tiling_on_sc=False, needs_layout_passes=False),
        scratch_types=[
            pltpu.VMEM((H, BW), jnp.int32),
            pltpu.VMEM((2, BW, D), jnp.float32),
            pltpu.VMEM((2, D, BW), jnp.float32),
            pltpu.SemaphoreType.DMA((2,)),
            pltpu.SemaphoreType.DMA((2,)),
            pltpu.SemaphoreType.DMA,
        ],
    )
    def gat(xT_hbm, tL_hbm, out_hbm, idx_v, rows_v, tr_v, gsem, osem, isem):
        wid = _worker_id()
        b0 = wid * BW

        pltpu.async_copy(xT_hbm.at[:, pl.ds(b0, BW)], idx_v, isem).wait()

        def gather(h, b):
            return pltpu.make_async_copy(
                tL_hbm.at[idx_v.at[h]], rows_v.at[b], gsem.at[b])

        def copy_out(h, b):
            return pltpu.make_async_copy(
                tr_v.at[b], out_hbm.at[h, :, pl.ds(b0, BW)], osem.at[b])

        gather(0, 0).start()
        iota = lax.iota(jnp.int32, _LANES)

        def body(h, carry):
            b = lax.rem(h, 2)
            nb = lax.rem(h + 1, 2)

            @pl.when(h + 1 < H)
            def _prefetch():
                gather(h + 1, nb).start()

            gather(h, b).wait()

            @pl.when(h >= 2)
            def _drain():
                copy_out(h - 2, b).wait()

            def cols(j, c2):
                r_idx = j * _LANES + iota
                for c in range(D):
                    c_idx = jnp.zeros((_LANES,), jnp.int32) + c
                    vals = plsc.load_gather(rows_v.at[b], [r_idx, c_idx])
                    tr_v[b, c, pl.ds(j * _LANES, _LANES)] = vals
                return c2

            lax.fori_loop(0, BW // _LANES, cols, 0)
            copy_out(h, b).start()
            return carry

        lax.fori_loop(0, H, body, 0)
        copy_out(H - 2, lax.rem(H - 2, 2)).wait()
        copy_out(H - 1, lax.rem(H - 1, 2)).wait()

    return gat(xT, tL)


def kernel(x, table):
    Bx, H = x.shape
    V, D = table.shape
    xT = jnp.swapaxes(x, 0, 1).astype(jnp.int32)
    tT = jnp.swapaxes(table, 0, 1)
    tL = _format_table(tT, blk=400, unroll=4)
    outT = _gather_t(xT, tL)
    return jnp.transpose(outT, (2, 0, 1))


# speedup vs baseline: 3.5885x; 3.5885x over previous
import functools

import jax
import jax.numpy as jnp
from jax import lax
from jax.experimental import pallas as pl
from jax.experimental.pallas import tpu as pltpu
from jax.experimental.pallas import tpu_sc as plsc

_NUM_CORES = 2
_NUM_SUBCORES = 16
_NUM_WORKERS = _NUM_CORES * _NUM_SUBCORES

_CHUNK = 1280
_FBLK = 512


def _tc_format(tT, *, blk=_FBLK):
    D, V = tT.shape
    nq = -(-(V // 4) // blk)
    Q = nq * blk

    def body(x0, x1, x2, x3, o_ref):
        for q, x in enumerate((x0, x1, x2, x3)):
            o_ref[:, pl.ds(q * D, D)] = x[...].T

    last_blk = -(-V // blk) - 1

    return pl.pallas_call(
        body,
        grid=(nq,),
        in_specs=[
            pl.BlockSpec((D, blk), functools.partial(
                lambda q, j: (0, jnp.minimum(q * nq + j, last_blk)), q))
            for q in range(4)
        ],
        out_specs=pl.BlockSpec((blk, 4 * D), lambda j: (j, 0)),
        out_shape=jax.ShapeDtypeStruct((Q, 4 * D), jnp.float32),
    )(tT, tT, tT, tT)


def _tc_unformat(packed, *, H, B):
    n4, D4 = packed.shape
    D = D4 // 4
    M = B // 4
    rows_per_blk = n4 // H

    def body(in_ref, o_ref):
        x = in_ref[...]
        for j in range(4):
            o_ref[0, :, pl.ds(j * M, M)] = x[:, j * D:(j + 1) * D].T

    return pl.pallas_call(
        body,
        grid=(H,),
        in_specs=[pl.BlockSpec((rows_per_blk, D4), lambda h: (h, 0))],
        out_specs=pl.BlockSpec((1, D, B), lambda h: (h, 0, 0)),
        out_shape=jax.ShapeDtypeStruct((H, D, B), jnp.float32),
    )(packed)


def _sc_gather(idx2d, tL, *, chunk):
    n_rows, chunk_ = idx2d.shape
    assert chunk_ == chunk
    V, D = tL.shape
    n_chunks = n_rows // _NUM_WORKERS
    B = n_rows * chunk

    mesh = plsc.VectorSubcoreMesh(core_axis_name="c", subcore_axis_name="s")

    @functools.partial(
        pl.kernel,
        out_type=jax.ShapeDtypeStruct((B, D), jnp.float32),
        mesh=mesh,
        compiler_params=pltpu.CompilerParams(use_tc_tiling_on_sc=False),
        scratch_types=[
            pltpu.VMEM((n_chunks, chunk), jnp.int32),
            pltpu.VMEM((2, chunk, D), jnp.float32),
            pltpu.SemaphoreType.DMA((2,)),
            pltpu.SemaphoreType.DMA((2,)),
            pltpu.SemaphoreType.DMA,
        ],
    )
    def emb(idx_hbm, table_hbm, out_hbm, idx_v, rows_v, gsem, osem, isem):
        wid = lax.axis_index("s") * _NUM_CORES + lax.axis_index("c")
        base = wid * n_chunks

        pltpu.async_copy(idx_hbm.at[pl.ds(base, n_chunks)], idx_v, isem).wait()

        def gather(i, b):
            return pltpu.make_async_copy(
                table_hbm.at[idx_v.at[i]], rows_v.at[b], gsem.at[b])

        def copy_out(i, b):
            return pltpu.make_async_copy(
                rows_v.at[b],
                out_hbm.at[pl.ds((base + i) * chunk, chunk)],
                osem.at[b])

        gather(0, 0).start()

        def body(i, carry):
            b = lax.rem(i, 2)
            nb = lax.rem(i + 1, 2)

            @pl.when(i + 1 < n_chunks)
            def _start_next():
                @pl.when(i >= 1)
                def _drain_prev_out():
                    copy_out(i - 1, nb).wait()
                gather(i + 1, nb).start()

            gather(i, b).wait()
            copy_out(i, b).start()
            return carry

        lax.fori_loop(0, n_chunks, body, 0)

        copy_out(n_chunks - 2, lax.rem(n_chunks - 2, 2)).wait()
        copy_out(n_chunks - 1, lax.rem(n_chunks - 1, 2)).wait()

    return emb(idx2d, tL)


def kernel(x, table):
    Bx, H = x.shape
    V, D = table.shape
    B = Bx * H
    M = Bx // 4
    QP = _FBLK * (-(-(V // 4) // _FBLK))

    tT = jnp.swapaxes(table, 0, 1)
    packed = _tc_format(tT)
    tL = packed.reshape(4 * QP, D)

    xp = x.reshape(4, M, H).transpose(2, 1, 0)
    xp = (xp % QP) * 4 + xp // QP
    idx2d = xp.reshape(B // _CHUNK, _CHUNK).astype(jnp.int32)

    lin = _sc_gather(idx2d, tL, chunk=_CHUNK)
    packed_out = lin.reshape(B // 4, 4 * D)
    outT = _tc_unformat(packed_out, H=H, B=Bx)
    return jnp.transpose(outT, (2, 0, 1))


# speedup vs baseline: 6.2507x; 1.7419x over previous
import functools

import jax
import jax.numpy as jnp
from jax import lax
from jax.experimental import pallas as pl
from jax.experimental.pallas import tpu as pltpu
from jax.experimental.pallas import tpu_sc as plsc

_NUM_CORES = 2
_NUM_SUBCORES = 16
_NUM_WORKERS = _NUM_CORES * _NUM_SUBCORES

_CHUNK = 1280
_FBLK = 1024


def _tc_format(tT, *, blk=_FBLK):
    D, V = tT.shape
    nq = -(-(V // 4) // blk)
    Q = nq * blk

    def body(x0, x1, x2, x3, o_ref):
        for q, x in enumerate((x0, x1, x2, x3)):
            o_ref[:, pl.ds(q * D, D)] = x[...].T

    last_blk = -(-V // blk) - 1

    return pl.pallas_call(
        body,
        grid=(nq,),
        in_specs=[
            pl.BlockSpec((D, blk), functools.partial(
                lambda q, j: (0, jnp.minimum(q * nq + j, last_blk)), q))
            for q in range(4)
        ],
        out_specs=pl.BlockSpec((blk, 4 * D), lambda j: (j, 0)),
        out_shape=jax.ShapeDtypeStruct((Q, 4 * D), jnp.float32),
    )(tT, tT, tT, tT)


def _tc_unformat(packed, *, H, B):
    n4, D4 = packed.shape
    D = D4 // 4
    M = B // 4
    rows_per_blk = n4 // H

    def body(in_ref, o_ref):
        x = in_ref[...]
        for j in range(4):
            o_ref[0, :, pl.ds(j * M, M)] = x[:, j * D:(j + 1) * D].T

    return pl.pallas_call(
        body,
        grid=(H,),
        in_specs=[pl.BlockSpec((rows_per_blk, D4), lambda h: (h, 0))],
        out_specs=pl.BlockSpec((1, D, B), lambda h: (h, 0, 0)),
        out_shape=jax.ShapeDtypeStruct((H, D, B), jnp.float32),
    )(packed)


def _sc_gather(idx2d, tL, *, chunk, H, M):
    n_rows, chunk_ = idx2d.shape
    assert chunk_ == chunk
    V, D = tL.shape
    n_chunks = n_rows // _NUM_WORKERS
    B = n_rows * chunk

    mesh = plsc.VectorSubcoreMesh(core_axis_name="c", subcore_axis_name="s")

    @functools.partial(
        pl.kernel,
        out_type=jax.ShapeDtypeStruct((B, D), jnp.float32),
        mesh=mesh,
        compiler_params=pltpu.CompilerParams(use_tc_tiling_on_sc=False),
        scratch_types=[
            pltpu.VMEM((n_chunks, chunk), jnp.int32),
            pltpu.VMEM((2, chunk, D), jnp.float32),
            pltpu.VMEM((2, chunk // 128, 128), jnp.int32),
            pltpu.SemaphoreType.DMA((2,)),
            pltpu.SemaphoreType.DMA((2,)),
            pltpu.SemaphoreType.DMA,
        ],
    )
    def emb(idx_hbm, table_hbm, out_hbm, idx_v, rows_v, dst_v, gsem, osem,
            isem):
        wid = lax.axis_index("s") * _NUM_CORES + lax.axis_index("c")
        base = wid * n_chunks

        pltpu.async_copy(idx_hbm.at[pl.ds(base, n_chunks)], idx_v, isem).wait()

        def gather(i, b):
            return pltpu.make_async_copy(
                table_hbm.at[idx_v.at[i]], rows_v.at[b], gsem.at[b])

        n_grp = chunk // 128

        def scatter_grp(b, g):
            return pltpu.make_async_copy(
                rows_v.at[b, pl.ds(g * 128, 128)],
                out_hbm.at[dst_v.at[b, g]], osem.at[b])

        def scatter_start(i, b):
            for g in range(n_grp):
                scatter_grp(b, g).start()

        def scatter_wait(i, b):
            for g in range(n_grp):
                scatter_grp(b, g).wait()

        lanes = lax.iota(jnp.int32, 16)

        def fill_dst(i, b):
            p0 = (base + i) * chunk

            def vec(k, carry):
                p = p0 + k * 16 + lanes
                h = lax.rem(p, H)
                bb = lax.div(p, H)
                m = lax.rem(bb, M)
                j = lax.div(bb, M)
                dst_v[b, lax.div(k, 8), pl.ds(lax.rem(k, 8) * 16, 16)] = (
                    h * (4 * M) + m * 4 + j)
                return carry

            lax.fori_loop(0, chunk // 16, vec, 0)

        gather(0, 0).start()
        fill_dst(0, 0)

        def body(i, carry):
            b = lax.rem(i, 2)
            nb = lax.rem(i + 1, 2)

            @pl.when(i + 1 < n_chunks)
            def _start_next():
                @pl.when(i >= 1)
                def _drain_prev_out():
                    scatter_wait(i - 1, nb)
                gather(i + 1, nb).start()
                fill_dst(i + 1, nb)

            gather(i, b).wait()
            scatter_start(i, b)
            return carry

        lax.fori_loop(0, n_chunks, body, 0)

        scatter_wait(n_chunks - 2, lax.rem(n_chunks - 2, 2))
        scatter_wait(n_chunks - 1, lax.rem(n_chunks - 1, 2))

    return emb(idx2d, tL)


def kernel(x, table):
    Bx, H = x.shape
    V, D = table.shape
    B = Bx * H
    M = Bx // 4
    QP = _FBLK * (-(-(V // 4) // _FBLK))

    tT = jnp.swapaxes(table, 0, 1)
    packed = _tc_format(tT)
    tL = packed.reshape(4 * QP, D)

    xv = (x % QP) * 4 + x // QP
    idx2d = xv.reshape(B // _CHUNK, _CHUNK).astype(jnp.int32)

    lin = _sc_gather(idx2d, tL, chunk=_CHUNK, H=H, M=M)
    packed_out = lin.reshape(B // 4, 4 * D)
    outT = _tc_unformat(packed_out, H=H, B=Bx)
    return jnp.transpose(outT, (2, 0, 1))
